# baseline (speedup 1.0000x reference)
import functools

import jax
import jax.numpy as jnp
from jax import lax
from jax.experimental import pallas as pl
from jax.experimental.pallas import tpu as pltpu
from jax.experimental.pallas import tpu_sc as plsc

VOCAB = 1_000_000
EMB_DIM = 64
TOTAL_TOKENS = 819_200

NUM_CORES = 2
NUM_SUBCORES = 16
NUM_WORKERS = NUM_CORES * NUM_SUBCORES
BPW = TOTAL_TOKENS // NUM_WORKERS
CHUNK = 256
NCHUNKS = BPW // CHUNK
NQUAD = NCHUNKS // 4
LANES = 16


def _build_gather():
    mesh = plsc.VectorSubcoreMesh(core_axis_name="c", subcore_axis_name="s")

    @functools.partial(
        pl.kernel,
        mesh=mesh,
        out_type=jax.ShapeDtypeStruct((EMB_DIM, TOTAL_TOKENS), jnp.float32),
        scratch_types=[
            pltpu.VMEM((CHUNK,), jnp.int32),
            pltpu.VMEM((CHUNK,), jnp.int32),
            pltpu.VMEM((CHUNK,), jnp.int32),
            pltpu.VMEM((CHUNK,), jnp.int32),
            pltpu.VMEM((CHUNK,), jnp.int32),
            pltpu.VMEM((CHUNK,), jnp.int32),
            pltpu.VMEM((CHUNK, 128), jnp.float32),
            pltpu.VMEM((CHUNK, 128), jnp.float32),
            pltpu.VMEM((EMB_DIM, CHUNK), jnp.float32),
            pltpu.VMEM((EMB_DIM, CHUNK), jnp.float32),
            pltpu.SemaphoreType.DMA,
            pltpu.SemaphoreType.DMA,
            pltpu.SemaphoreType.DMA,
            pltpu.SemaphoreType.DMA,
            pltpu.SemaphoreType.DMA,
        ],
        compiler_params=pltpu.CompilerParams(needs_layout_passes=False),
    )
    def emb_gather(tbl_hbm, idx_hbm, out_hbm,
                   idx0, idx1, idx2, idx3, gidx0, gidx1,
                   rows0, rows1, slab0, slab1,
                   isem, gsem0, gsem1, osem0, osem1):
        wid = lax.axis_index("s") * NUM_CORES + lax.axis_index("c")
        base = wid * BPW
        idxs = (idx0, idx1, idx2, idx3)
        gidxs = (gidx0, gidx1)
        rows = (rows0, rows1)
        slabs = (slab0, slab1)
        gsems = (gsem0, gsem1)
        osems = (osem0, osem1)

        def idx_copy(c, slot):
            return pltpu.make_async_copy(
                idx_hbm.at[pl.ds(base + c * CHUNK, CHUNK)], idxs[slot], isem)

        def start_gather(slot, p):
            src = idxs[slot]
            for j in range(CHUNK // LANES):
                sl = pl.ds(j * LANES, LANES)
                gidxs[p][sl] = lax.shift_right_logical(src[sl], 1)
            pltpu.async_copy(tbl_hbm.at[gidxs[p]], rows[p], gsems[p])

        def wait_gather(p):
            pltpu.make_async_copy(
                tbl_hbm.at[gidxs[p]], rows[p], gsems[p]).wait()

        def slab_copy(c, p):
            return pltpu.make_async_copy(
                slabs[p],
                out_hbm.at[:, pl.ds(base + c * CHUNK, CHUNK)], osems[p])

        def transpose_chunk(slot, p):
            src = idxs[slot]
            lane = lax.broadcasted_iota(jnp.int32, (LANES,), 0)

            def grp(g, carry):
                sl = pl.ds(g * LANES, LANES)
                half = lax.shift_left(
                    lax.bitwise_and(src[sl], jnp.int32(1)), 6)
                lvec = lane + g * LANES
                for d in range(EMB_DIM):
                    vals = plsc.load_gather(
                        rows[p], [lvec, half + jnp.int32(d)])
                    slabs[p][d, sl] = vals
                return carry

            lax.fori_loop(0, CHUNK // LANES, grp, 0)

        def phase(c, k, first):
            p = k % 2

            @pl.when(c + 1 < NCHUNKS)
            def _():
                idx_copy(c + 1, (k + 1) % 4).wait()
                start_gather((k + 1) % 4, 1 - p)

            @pl.when(c + 2 < NCHUNKS)
            def _():
                idx_copy(c + 2, (k + 2) % 4).start()

            wait_gather(p)
            if not first:
                @pl.when(c >= 2)
                def _():
                    slab_copy(c - 2, p).wait()

            transpose_chunk(k, p)
            slab_copy(c, p).start()

        idx_copy(0, 0).start()
        idx_copy(0, 0).wait()
        start_gather(0, 0)
        idx_copy(1, 1).start()

        def body(i, carry):
            c0 = 4 * i
            for k in range(4):
                phase(c0 + k, k, first=False)
            return carry

        for k in range(4):
            phase(jnp.int32(k), k, first=k < 2)
        lax.fori_loop(1, NQUAD, body, 0)
        slab_copy(NCHUNKS - 2, 0).wait()
        slab_copy(NCHUNKS - 1, 1).wait()

    return emb_gather


_emb_gather = _build_gather()


def kernel(input, batch_sizes, table):
    tbl2 = table.reshape(VOCAB // 2, 2 * EMB_DIM)
    out_t = _emb_gather(tbl2, input)
    return (out_t.T, batch_sizes)


# speedup vs baseline: 1.2238x; 1.2238x over previous
import functools

import jax
import jax.numpy as jnp
from jax import lax
from jax.experimental import pallas as pl
from jax.experimental.pallas import tpu as pltpu
from jax.experimental.pallas import tpu_sc as plsc

VOCAB = 1_000_000
EMB_DIM = 64
TOTAL_TOKENS = 819_200

NUM_CORES = 2
NUM_SUBCORES = 16
NUM_WORKERS = NUM_CORES * NUM_SUBCORES
BPW = TOTAL_TOKENS // NUM_WORKERS
CHUNK = 256
NCHUNKS = BPW // CHUNK
NQUAD = NCHUNKS // 4
LANES = 16


def _build_gather():
    mesh = plsc.VectorSubcoreMesh(core_axis_name="c", subcore_axis_name="s")

    @functools.partial(
        pl.kernel,
        mesh=mesh,
        out_type=jax.ShapeDtypeStruct((EMB_DIM, TOTAL_TOKENS), jnp.float32),
        scratch_types=[
            pltpu.VMEM((CHUNK,), jnp.int32),
            pltpu.VMEM((CHUNK,), jnp.int32),
            pltpu.VMEM((CHUNK,), jnp.int32),
            pltpu.VMEM((CHUNK,), jnp.int32),
            pltpu.VMEM((CHUNK,), jnp.int32),
            pltpu.VMEM((CHUNK,), jnp.int32),
            pltpu.VMEM((CHUNK, 128), jnp.float32),
            pltpu.VMEM((CHUNK, 128), jnp.float32),
            pltpu.VMEM((EMB_DIM, CHUNK), jnp.float32),
            pltpu.VMEM((EMB_DIM, CHUNK), jnp.float32),
            pltpu.SemaphoreType.DMA,
            pltpu.SemaphoreType.DMA,
            pltpu.SemaphoreType.DMA,
            pltpu.SemaphoreType.DMA,
            pltpu.SemaphoreType.DMA,
        ],
        compiler_params=pltpu.CompilerParams(needs_layout_passes=False),
    )
    def emb_gather(tbl_hbm, idx_hbm, out_hbm,
                   idx0, idx1, idx2, idx3, gidx0, gidx1,
                   rows0, rows1, slab0, slab1,
                   isem, gsem0, gsem1, osem0, osem1):
        wid = lax.axis_index("s") * NUM_CORES + lax.axis_index("c")
        base = wid * BPW
        idxs = (idx0, idx1, idx2, idx3)
        gidxs = (gidx0, gidx1)
        rows = (rows0, rows1)
        slabs = (slab0, slab1)
        gsems = (gsem0, gsem1)
        osems = (osem0, osem1)

        def idx_copy(c, slot):
            return pltpu.make_async_copy(
                idx_hbm.at[pl.ds(base + c * CHUNK, CHUNK)], idxs[slot], isem)

        def start_gather(slot, p):
            src = idxs[slot]
            for j in range(CHUNK // LANES):
                sl = pl.ds(j * LANES, LANES)
                gidxs[p][sl] = lax.shift_right_logical(src[sl], 1)
            pltpu.async_copy(tbl_hbm.at[gidxs[p]], rows[p], gsems[p])

        def wait_gather(p):
            pltpu.make_async_copy(
                tbl_hbm.at[gidxs[p]], rows[p], gsems[p]).wait()

        def slab_copy(c, p):
            return pltpu.make_async_copy(
                slabs[p],
                out_hbm.at[:, pl.ds(base + c * CHUNK, CHUNK)], osems[p])

        def transpose_chunk(slot, p):
            src = idxs[slot]
            lane = lax.broadcasted_iota(jnp.int32, (LANES,), 0)

            @plsc.parallel_loop(0, CHUNK // LANES, 1, unroll=4)
            def grp(g):
                sl = pl.ds(g * LANES, LANES)
                half = lax.shift_left(
                    lax.bitwise_and(src[sl], jnp.int32(1)), 6)
                lvec = lane + g * LANES
                for d in range(EMB_DIM):
                    vals = plsc.load_gather(
                        rows[p], [lvec, half + jnp.int32(d)])
                    slabs[p][d, sl] = vals

        def phase(c, k, first):
            p = k % 2

            @pl.when(c + 1 < NCHUNKS)
            def _():
                idx_copy(c + 1, (k + 1) % 4).wait()
                start_gather((k + 1) % 4, 1 - p)

            @pl.when(c + 2 < NCHUNKS)
            def _():
                idx_copy(c + 2, (k + 2) % 4).start()

            wait_gather(p)
            if not first:
                @pl.when(c >= 2)
                def _():
                    slab_copy(c - 2, p).wait()

            transpose_chunk(k, p)
            slab_copy(c, p).start()

        idx_copy(0, 0).start()
        idx_copy(0, 0).wait()
        start_gather(0, 0)
        idx_copy(1, 1).start()

        def body(i, carry):
            c0 = 4 * i
            for k in range(4):
                phase(c0 + k, k, first=False)
            return carry

        for k in range(4):
            phase(jnp.int32(k), k, first=k < 2)
        lax.fori_loop(1, NQUAD, body, 0)
        slab_copy(NCHUNKS - 2, 0).wait()
        slab_copy(NCHUNKS - 1, 1).wait()

    return emb_gather


_emb_gather = _build_gather()


def kernel(input, batch_sizes, table):
    tbl2 = table.reshape(VOCAB // 2, 2 * EMB_DIM)
    out_t = _emb_gather(tbl2, input)
    return (out_t.T, batch_sizes)


# speedup vs baseline: 1.6309x; 1.3326x over previous
import functools

import jax
import jax.numpy as jnp
from jax import lax
from jax.experimental import pallas as pl
from jax.experimental.pallas import tpu as pltpu
from jax.experimental.pallas import tpu_sc as plsc

VOCAB = 1_000_000
EMB_DIM = 64
TOTAL_TOKENS = 819_200

NUM_CORES = 2
NUM_SUBCORES = 16
NUM_WORKERS = NUM_CORES * NUM_SUBCORES
BPW = TOTAL_TOKENS // NUM_WORKERS
CHUNK = 256
NCHUNKS = BPW // CHUNK
NQUAD = NCHUNKS // 4
LANES = 16


def _build_gather():
    mesh = plsc.VectorSubcoreMesh(core_axis_name="c", subcore_axis_name="s")

    @functools.partial(
        pl.kernel,
        mesh=mesh,
        out_type=jax.ShapeDtypeStruct((EMB_DIM, TOTAL_TOKENS), jnp.float32),
        scratch_types=[
            pltpu.VMEM((CHUNK,), jnp.int32),
            pltpu.VMEM((CHUNK,), jnp.int32),
            pltpu.VMEM((CHUNK,), jnp.int32),
            pltpu.VMEM((CHUNK,), jnp.int32),
            pltpu.VMEM((CHUNK,), jnp.int32),
            pltpu.VMEM((CHUNK,), jnp.int32),
            pltpu.VMEM((CHUNK, 128), jnp.float32),
            pltpu.VMEM((CHUNK, 128), jnp.float32),
            pltpu.VMEM((EMB_DIM, CHUNK), jnp.float32),
            pltpu.VMEM((EMB_DIM, CHUNK), jnp.float32),
            pltpu.SemaphoreType.DMA,
            pltpu.SemaphoreType.DMA,
            pltpu.SemaphoreType.DMA,
            pltpu.SemaphoreType.DMA,
            pltpu.SemaphoreType.DMA,
        ],
        compiler_params=pltpu.CompilerParams(needs_layout_passes=False),
    )
    def emb_gather(tbl_hbm, idx_hbm, out_hbm,
                   idx0, idx1, idx2, idx3, gidx0, gidx1,
                   rows0, rows1, slab0, slab1,
                   isem, gsem0, gsem1, osem0, osem1):
        wid = lax.axis_index("s") * NUM_CORES + lax.axis_index("c")
        base = wid * BPW
        idxs = (idx0, idx1, idx2, idx3)
        gidxs = (gidx0, gidx1)
        rows = (rows0, rows1)
        slabs = (slab0, slab1)
        gsems = (gsem0, gsem1)
        osems = (osem0, osem1)

        def idx_copy(c, slot):
            return pltpu.make_async_copy(
                idx_hbm.at[pl.ds(base + c * CHUNK, CHUNK)], idxs[slot], isem)

        def start_gather(slot, p):
            src = idxs[slot]
            for j in range(CHUNK // LANES):
                sl = pl.ds(j * LANES, LANES)
                gidxs[p][sl] = lax.shift_right_logical(src[sl], 1)
            pltpu.async_copy(tbl_hbm.at[gidxs[p]], rows[p], gsems[p])

        def wait_gather(p):
            pltpu.make_async_copy(
                tbl_hbm.at[gidxs[p]], rows[p], gsems[p]).wait()

        def slab_copy(c, p):
            return pltpu.make_async_copy(
                slabs[p],
                out_hbm.at[:, pl.ds(base + c * CHUNK, CHUNK)], osems[p])

        def transpose_chunk(slot, p):
            src = idxs[slot]
            lane = lax.broadcasted_iota(jnp.int32, (LANES,), 0)

            for g in range(CHUNK // LANES):
                sl = pl.ds(g * LANES, LANES)
                half = lax.shift_left(
                    lax.bitwise_and(src[sl], jnp.int32(1)), 6)
                lvec = lane + g * LANES

                zero = lane * 0

                @plsc.parallel_loop(0, EMB_DIM, 1, unroll=8)
                def dloop(d):
                    vals = plsc.load_gather(rows[p], [lvec, half + d])
                    plsc.store_scatter(slabs[p], [zero + d, lvec], vals)

        def phase(c, k, first):
            p = k % 2

            @pl.when(c + 1 < NCHUNKS)
            def _():
                idx_copy(c + 1, (k + 1) % 4).wait()
                start_gather((k + 1) % 4, 1 - p)

            @pl.when(c + 2 < NCHUNKS)
            def _():
                idx_copy(c + 2, (k + 2) % 4).start()

            wait_gather(p)
            if not first:
                @pl.when(c >= 2)
                def _():
                    slab_copy(c - 2, p).wait()

            transpose_chunk(k, p)
            slab_copy(c, p).start()

        idx_copy(0, 0).start()
        idx_copy(0, 0).wait()
        start_gather(0, 0)
        idx_copy(1, 1).start()

        def body(i, carry):
            c0 = 4 * i
            for k in range(4):
                phase(c0 + k, k, first=False)
            return carry

        for k in range(4):
            phase(jnp.int32(k), k, first=k < 2)
        lax.fori_loop(1, NQUAD, body, 0)
        slab_copy(NCHUNKS - 2, 0).wait()
        slab_copy(NCHUNKS - 1, 1).wait()

    return emb_gather


_emb_gather = _build_gather()


def kernel(input, batch_sizes, table):
    tbl2 = table.reshape(VOCAB // 2, 2 * EMB_DIM)
    out_t = _emb_gather(tbl2, input)
    return (out_t.T, batch_sizes)


# speedup vs baseline: 2.1224x; 1.3014x over previous
import functools

import jax
import jax.numpy as jnp
from jax import lax
from jax.experimental import pallas as pl
from jax.experimental.pallas import tpu as pltpu
from jax.experimental.pallas import tpu_sc as plsc

VOCAB = 1_000_000
EMB_DIM = 64
TOTAL_TOKENS = 819_200

NUM_CORES = 2
NUM_SUBCORES = 16
NUM_WORKERS = NUM_CORES * NUM_SUBCORES
BPW = TOTAL_TOKENS // NUM_WORKERS
CHUNK = 256
NCHUNKS = BPW // CHUNK
NQUAD = NCHUNKS // 4
LANES = 16


def _build_gather():
    mesh = plsc.VectorSubcoreMesh(core_axis_name="c", subcore_axis_name="s")

    @functools.partial(
        pl.kernel,
        mesh=mesh,
        out_type=jax.ShapeDtypeStruct((EMB_DIM, TOTAL_TOKENS), jnp.float32),
        scratch_types=[
            pltpu.VMEM((CHUNK,), jnp.int32),
            pltpu.VMEM((CHUNK,), jnp.int32),
            pltpu.VMEM((CHUNK,), jnp.int32),
            pltpu.VMEM((CHUNK,), jnp.int32),
            pltpu.VMEM((CHUNK,), jnp.int32),
            pltpu.VMEM((CHUNK,), jnp.int32),
            pltpu.VMEM((CHUNK, 128), jnp.float32),
            pltpu.VMEM((CHUNK, 128), jnp.float32),
            pltpu.VMEM((EMB_DIM, CHUNK), jnp.float32),
            pltpu.VMEM((EMB_DIM, CHUNK), jnp.float32),
            pltpu.SemaphoreType.DMA,
            pltpu.SemaphoreType.DMA,
            pltpu.SemaphoreType.DMA,
            pltpu.SemaphoreType.DMA,
            pltpu.SemaphoreType.DMA,
        ],
        compiler_params=pltpu.CompilerParams(needs_layout_passes=False),
    )
    def emb_gather(tbl_hbm, idx_hbm, out_hbm,
                   idx0, idx1, idx2, idx3, gidx0, gidx1,
                   rows0, rows1, slab0, slab1,
                   isem, gsem0, gsem1, osem0, osem1):
        wid = lax.axis_index("s") * NUM_CORES + lax.axis_index("c")
        base = wid * BPW
        idxs = (idx0, idx1, idx2, idx3)
        gidxs = (gidx0, gidx1)
        rows = (rows0, rows1)
        slabs = (slab0, slab1)
        gsems = (gsem0, gsem1)
        osems = (osem0, osem1)

        def idx_copy(c, slot):
            return pltpu.make_async_copy(
                idx_hbm.at[pl.ds(base + c * CHUNK, CHUNK)], idxs[slot], isem)

        def start_gather(slot, p):
            src = idxs[slot]
            for j in range(CHUNK // LANES):
                sl = pl.ds(j * LANES, LANES)
                gidxs[p][sl] = lax.shift_right_logical(src[sl], 1)
            pltpu.async_copy(tbl_hbm.at[gidxs[p]], rows[p], gsems[p])

        def wait_gather(p):
            pltpu.make_async_copy(
                tbl_hbm.at[gidxs[p]], rows[p], gsems[p]).wait()

        def slab_copy(c, p):
            return pltpu.make_async_copy(
                slabs[p],
                out_hbm.at[:, pl.ds(base + c * CHUNK, CHUNK)], osems[p])

        def transpose_chunk(slot, p):
            src = idxs[slot]
            lane = lax.broadcasted_iota(jnp.int32, (LANES,), 0)

            for g in range(CHUNK // LANES):
                sl = pl.ds(g * LANES, LANES)
                half = lax.shift_left(
                    lax.bitwise_and(src[sl], jnp.int32(1)), 6)
                lvec = lane + g * LANES

                @plsc.parallel_loop(0, EMB_DIM, 1, unroll=8)
                def dloop(d):
                    cvec = lax.bitwise_and(lane + d, jnp.int32(EMB_DIM - 1))
                    vals = plsc.load_gather(rows[p], [lvec, half + cvec])
                    plsc.store_scatter(slabs[p], [cvec, lvec], vals)

        def phase(c, k, first):
            p = k % 2

            @pl.when(c + 1 < NCHUNKS)
            def _():
                idx_copy(c + 1, (k + 1) % 4).wait()
                start_gather((k + 1) % 4, 1 - p)

            @pl.when(c + 2 < NCHUNKS)
            def _():
                idx_copy(c + 2, (k + 2) % 4).start()

            wait_gather(p)
            if not first:
                @pl.when(c >= 2)
                def _():
                    slab_copy(c - 2, p).wait()

            transpose_chunk(k, p)
            slab_copy(c, p).start()

        idx_copy(0, 0).start()
        idx_copy(0, 0).wait()
        start_gather(0, 0)
        idx_copy(1, 1).start()

        def body(i, carry):
            c0 = 4 * i
            for k in range(4):
                phase(c0 + k, k, first=False)
            return carry

        for k in range(4):
            phase(jnp.int32(k), k, first=k < 2)
        lax.fori_loop(1, NQUAD, body, 0)
        slab_copy(NCHUNKS - 2, 0).wait()
        slab_copy(NCHUNKS - 1, 1).wait()

    return emb_gather


_emb_gather = _build_gather()


def kernel(input, batch_sizes, table):
    tbl2 = table.reshape(VOCAB // 2, 2 * EMB_DIM)
    out_t = _emb_gather(tbl2, input)
    return (out_t.T, batch_sizes)


# speedup vs baseline: 3.5679x; 1.6811x over previous
import functools

import jax
import jax.numpy as jnp
from jax import lax
from jax.experimental import pallas as pl
from jax.experimental.pallas import tpu as pltpu
from jax.experimental.pallas import tpu_sc as plsc

VOCAB = 1_000_000
EMB_DIM = 64
TOTAL_TOKENS = 819_200

NUM_CORES = 2
NUM_SUBCORES = 16
NUM_WORKERS = NUM_CORES * NUM_SUBCORES
BPW = TOTAL_TOKENS // NUM_WORKERS
CHUNK = 256
NCHUNKS = BPW // CHUNK
NQUAD = NCHUNKS // 4
LANES = 16

FULL_TCOLS = VOCAB // 128
COLS_PW = FULL_TCOLS // NUM_WORKERS
EXTRA_COLS = FULL_TCOLS - COLS_PW * NUM_WORKERS
TAIL_ROWS = VOCAB - FULL_TCOLS * 128


def _build_phase1():
    mesh = plsc.VectorSubcoreMesh(core_axis_name="c", subcore_axis_name="s")

    @functools.partial(
        pl.kernel,
        mesh=mesh,
        out_type=jax.ShapeDtypeStruct((VOCAB // 2, 128), jnp.float32),
        scratch_types=[
            pltpu.VMEM((EMB_DIM, 128), jnp.float32),
            pltpu.VMEM((EMB_DIM, 128), jnp.float32),
            pltpu.VMEM((EMB_DIM, 128), jnp.float32),
            pltpu.VMEM((EMB_DIM, 128), jnp.float32),
            pltpu.SemaphoreType.DMA,
            pltpu.SemaphoreType.DMA,
            pltpu.SemaphoreType.DMA,
            pltpu.SemaphoreType.DMA,
        ],
        compiler_params=pltpu.CompilerParams(needs_layout_passes=False),
    )
    def relayout(tt_hbm, tail_hbm, out_hbm, in0, in1, ot0, ot1,
                 isem0, isem1, osem0, osem1):
        wid = lax.axis_index("s") * NUM_CORES + lax.axis_index("c")
        base = wid * COLS_PW
        ins = (in0, in1)
        ots = (ot0, ot1)
        isems = (isem0, isem1)
        osems = (osem0, osem1)
        lane = lax.broadcasted_iota(jnp.int32, (LANES,), 0)

        def in_copy(c, p):
            return pltpu.make_async_copy(
                tt_hbm.at[:, pl.ds(c * 128, 128)], ins[p], isems[p])

        def out_copy(c, p):
            return pltpu.make_async_copy(
                ots[p], out_hbm.at[pl.ds(c * 64, 64), :], osems[p])

        def transpose_block(p):
            src, dst = ins[p], ots[p]
            for d0 in range(0, EMB_DIM, LANES):
                dvec = lane + d0

                @plsc.parallel_loop(0, 128, 1, unroll=8)
                def lloop(l0):
                    lvec = lax.bitwise_and(lane + l0, jnp.int32(127))
                    ivec = lax.shift_right_logical(lvec, 1)
                    jvec = lax.bitwise_or(
                        lax.shift_left(
                            lax.bitwise_and(lvec, jnp.int32(1)), 6), dvec)
                    vals = plsc.load_gather(src, [dvec, lvec])
                    plsc.store_scatter(dst, [ivec, jvec], vals)

        def run_col(c, p, first, last):
            in_copy(c, p).wait()
            transpose_block(p)

            @pl.when(c + 2 < base + COLS_PW)
            def _():
                in_copy(c + 2, p).start()

            if not first:
                out_copy(c - 2, p).wait()
            out_copy(c, p).start()
            _ = last

        in_copy(base, 0).start()
        in_copy(base + 1, 1).start()

        def body(i, carry):
            c0 = base + 2 * i
            run_col(c0, 0, first=False, last=False)
            run_col(c0 + 1, 1, first=False, last=False)
            return carry

        run_col(base, 0, first=True, last=False)
        run_col(base + 1, 1, first=True, last=False)
        lax.fori_loop(1, COLS_PW // 2, body, 0)
        out_copy(base + COLS_PW - 2, 0).wait()
        out_copy(base + COLS_PW - 1, 1).wait()

        @pl.when(wid < EXTRA_COLS)
        def _():
            c = FULL_TCOLS - EXTRA_COLS + wid
            pltpu.async_copy(
                tt_hbm.at[:, pl.ds(c * 128, 128)], in0, isem0).wait()
            transpose_block(0)
            pltpu.async_copy(
                ot0, out_hbm.at[pl.ds(c * 64, 64), :], osem0).wait()

        @pl.when(wid == NUM_WORKERS - 1)
        def _():
            pltpu.async_copy(
                tail_hbm, in1.at[pl.ds(0, 32), :], isem1).wait()
            pltpu.async_copy(
                in1.at[pl.ds(0, 32), :],
                out_hbm.at[pl.ds(VOCAB // 2 - 32, 32), :], osem1).wait()

    return relayout


def _build_phase2():
    mesh = plsc.VectorSubcoreMesh(core_axis_name="c", subcore_axis_name="s")

    @functools.partial(
        pl.kernel,
        mesh=mesh,
        out_type=jax.ShapeDtypeStruct((EMB_DIM, TOTAL_TOKENS), jnp.float32),
        scratch_types=[
            pltpu.VMEM((CHUNK,), jnp.int32),
            pltpu.VMEM((CHUNK,), jnp.int32),
            pltpu.VMEM((CHUNK,), jnp.int32),
            pltpu.VMEM((CHUNK,), jnp.int32),
            pltpu.VMEM((CHUNK,), jnp.int32),
            pltpu.VMEM((CHUNK,), jnp.int32),
            pltpu.VMEM((CHUNK, 128), jnp.float32),
            pltpu.VMEM((CHUNK, 128), jnp.float32),
            pltpu.VMEM((EMB_DIM, CHUNK), jnp.float32),
            pltpu.VMEM((EMB_DIM, CHUNK), jnp.float32),
            pltpu.SemaphoreType.DMA,
            pltpu.SemaphoreType.DMA,
            pltpu.SemaphoreType.DMA,
            pltpu.SemaphoreType.DMA,
            pltpu.SemaphoreType.DMA,
        ],
        compiler_params=pltpu.CompilerParams(needs_layout_passes=False),
    )
    def emb_gather(tbl_hbm, idx_hbm, out_hbm,
                   idx0, idx1, idx2, idx3, gidx0, gidx1,
                   rows0, rows1, slab0, slab1,
                   isem, gsem0, gsem1, osem0, osem1):
        wid = lax.axis_index("s") * NUM_CORES + lax.axis_index("c")
        base = wid * BPW
        idxs = (idx0, idx1, idx2, idx3)
        gidxs = (gidx0, gidx1)
        rows = (rows0, rows1)
        slabs = (slab0, slab1)
        gsems = (gsem0, gsem1)
        osems = (osem0, osem1)
        lane = lax.broadcasted_iota(jnp.int32, (LANES,), 0)

        def idx_copy(c, slot):
            return pltpu.make_async_copy(
                idx_hbm.at[pl.ds(base + c * CHUNK, CHUNK)], idxs[slot], isem)

        def start_gather(slot, p):
            src = idxs[slot]
            for j in range(CHUNK // LANES):
                sl = pl.ds(j * LANES, LANES)
                gidxs[p][sl] = lax.shift_right_logical(src[sl], 1)
            pltpu.async_copy(tbl_hbm.at[gidxs[p]], rows[p], gsems[p])

        def wait_gather(p):
            pltpu.make_async_copy(
                tbl_hbm.at[gidxs[p]], rows[p], gsems[p]).wait()

        def slab_copy(c, p):
            return pltpu.make_async_copy(
                slabs[p],
                out_hbm.at[:, pl.ds(base + c * CHUNK, CHUNK)], osems[p])

        def transpose_chunk(slot, p):
            src = idxs[slot]
            for g in range(CHUNK // LANES):
                sl = pl.ds(g * LANES, LANES)
                half = lax.shift_left(
                    lax.bitwise_and(src[sl], jnp.int32(1)), 6)
                lvec = lane + g * LANES

                @plsc.parallel_loop(0, EMB_DIM, 1, unroll=8)
                def dloop(d):
                    cvec = lax.bitwise_and(lane + d, jnp.int32(EMB_DIM - 1))
                    vals = plsc.load_gather(rows[p], [lvec, half + cvec])
                    plsc.store_scatter(slabs[p], [cvec, lvec], vals)

        def phase(c, k, first):
            p = k % 2

            @pl.when(c + 1 < NCHUNKS)
            def _():
                idx_copy(c + 1, (k + 1) % 4).wait()
                start_gather((k + 1) % 4, 1 - p)

            @pl.when(c + 2 < NCHUNKS)
            def _():
                idx_copy(c + 2, (k + 2) % 4).start()

            wait_gather(p)
            if not first:
                @pl.when(c >= 2)
                def _():
                    slab_copy(c - 2, p).wait()

            transpose_chunk(k, p)
            slab_copy(c, p).start()

        idx_copy(0, 0).start()
        idx_copy(0, 0).wait()
        start_gather(0, 0)
        idx_copy(1, 1).start()

        def body(i, carry):
            c0 = 4 * i
            for k in range(4):
                phase(c0 + k, k, first=False)
            return carry

        for k in range(4):
            phase(jnp.int32(k), k, first=k < 2)
        lax.fori_loop(1, NQUAD, body, 0)
        slab_copy(NCHUNKS - 2, 0).wait()
        slab_copy(NCHUNKS - 1, 1).wait()

    return emb_gather


_relayout = _build_phase1()
_emb_gather = _build_phase2()


def kernel(input, batch_sizes, table):
    table_t = table.T
    tail2 = table[VOCAB - TAIL_ROWS:].reshape(TAIL_ROWS // 2, 128)
    tbl2 = _relayout(table_t, tail2)
    out_t = _emb_gather(tbl2, input)
    return (out_t.T, batch_sizes)


# speedup vs baseline: 3.6347x; 1.0187x over previous
import functools

import jax
import jax.numpy as jnp
from jax import lax
from jax.experimental import pallas as pl
from jax.experimental.pallas import tpu as pltpu
from jax.experimental.pallas import tpu_sc as plsc

VOCAB = 1_000_000
EMB_DIM = 64
TOTAL_TOKENS = 819_200

NUM_CORES = 2
NUM_SUBCORES = 16
NUM_WORKERS = NUM_CORES * NUM_SUBCORES
BPW = TOTAL_TOKENS // NUM_WORKERS
CHUNK = 256
NCHUNKS = BPW // CHUNK
NQUAD = NCHUNKS // 4
LANES = 16

FULL_STEPS = VOCAB // 256
STEPS_PW = FULL_STEPS // NUM_WORKERS
EXTRA_STEPS = FULL_STEPS - STEPS_PW * NUM_WORKERS
TAIL_ROWS = VOCAB - FULL_STEPS * 256


def _build_phase1():
    mesh = plsc.VectorSubcoreMesh(core_axis_name="c", subcore_axis_name="s")

    @functools.partial(
        pl.kernel,
        mesh=mesh,
        out_type=jax.ShapeDtypeStruct((VOCAB // 2, 128), jnp.float32),
        scratch_types=[
            pltpu.VMEM((EMB_DIM, 256), jnp.float32),
            pltpu.VMEM((EMB_DIM, 256), jnp.float32),
            pltpu.VMEM((128, 128), jnp.float32),
            pltpu.VMEM((128, 128), jnp.float32),
            pltpu.SemaphoreType.DMA,
            pltpu.SemaphoreType.DMA,
            pltpu.SemaphoreType.DMA,
            pltpu.SemaphoreType.DMA,
        ],
        compiler_params=pltpu.CompilerParams(needs_layout_passes=False),
    )
    def relayout(tt_hbm, tail_hbm, out_hbm, in0, in1, ot0, ot1,
                 isem0, isem1, osem0, osem1):
        wid = lax.axis_index("s") * NUM_CORES + lax.axis_index("c")
        base = wid * STEPS_PW
        ins = (in0, in1)
        ots = (ot0, ot1)
        isems = (isem0, isem1)
        osems = (osem0, osem1)
        lane = lax.broadcasted_iota(jnp.int32, (LANES,), 0)

        def in_copy(c, p, width=256):
            return pltpu.make_async_copy(
                tt_hbm.at[:, pl.ds(c * 256, width)],
                ins[p].at[:, pl.ds(0, width)], isems[p])

        def out_copy(c, p, rows_n=128):
            return pltpu.make_async_copy(
                ots[p].at[pl.ds(0, rows_n), :],
                out_hbm.at[pl.ds(c * 128, rows_n), :], osems[p])

        def transpose_block(p, width=256):
            src, dst = ins[p], ots[p]
            for d0 in range(0, EMB_DIM, LANES):
                dvec = lane + d0

                @plsc.parallel_loop(0, width, 1, unroll=8)
                def lloop(l0):
                    lvec = lax.bitwise_and(lane + l0, jnp.int32(width - 1))
                    ivec = lax.shift_right_logical(lvec, 1)
                    jvec = lax.bitwise_or(
                        lax.shift_left(
                            lax.bitwise_and(lvec, jnp.int32(1)), 6), dvec)
                    vals = plsc.load_gather(src, [dvec, lvec])
                    plsc.store_scatter(dst, [ivec, jvec], vals)

        def run_col(c, p, first):
            in_copy(c, p).wait()
            transpose_block(p)

            @pl.when(c + 2 < base + STEPS_PW)
            def _():
                in_copy(c + 2, p).start()

            if not first:
                out_copy(c - 2, p).wait()
            out_copy(c, p).start()

        in_copy(base, 0).start()
        in_copy(base + 1, 1).start()

        def body(i, carry):
            c0 = base + 2 * i
            run_col(c0, 0, first=False)
            run_col(c0 + 1, 1, first=False)
            return carry

        run_col(base, 0, first=True)
        run_col(base + 1, 1, first=True)
        lax.fori_loop(1, STEPS_PW // 2, body, 0)
        out_copy(base + STEPS_PW - 2, 0).wait()
        out_copy(base + STEPS_PW - 1, 1).wait()

        @pl.when(wid < EXTRA_STEPS)
        def _():
            c = FULL_STEPS - EXTRA_STEPS + wid
            pltpu.async_copy(
                tt_hbm.at[:, pl.ds(c * 256, 256)], in0, isem0).wait()
            transpose_block(0)
            pltpu.async_copy(
                ot0, out_hbm.at[pl.ds(c * 128, 128), :], osem0).wait()

        @pl.when(wid == NUM_WORKERS - 1)
        def _():
            pltpu.async_copy(
                tail_hbm, in1.at[pl.ds(0, 32), pl.ds(0, 128)], isem1).wait()
            pltpu.async_copy(
                in1.at[pl.ds(0, 32), pl.ds(0, 128)],
                out_hbm.at[pl.ds(VOCAB // 2 - 32, 32), :], osem1).wait()

    return relayout


def _build_phase2():
    mesh = plsc.VectorSubcoreMesh(core_axis_name="c", subcore_axis_name="s")

    @functools.partial(
        pl.kernel,
        mesh=mesh,
        out_type=jax.ShapeDtypeStruct((EMB_DIM, TOTAL_TOKENS), jnp.float32),
        scratch_types=[
            pltpu.VMEM((CHUNK,), jnp.int32),
            pltpu.VMEM((CHUNK,), jnp.int32),
            pltpu.VMEM((CHUNK,), jnp.int32),
            pltpu.VMEM((CHUNK,), jnp.int32),
            pltpu.VMEM((CHUNK,), jnp.int32),
            pltpu.VMEM((CHUNK,), jnp.int32),
            pltpu.VMEM((CHUNK, 128), jnp.float32),
            pltpu.VMEM((CHUNK, 128), jnp.float32),
            pltpu.VMEM((EMB_DIM, CHUNK), jnp.float32),
            pltpu.VMEM((EMB_DIM, CHUNK), jnp.float32),
            pltpu.SemaphoreType.DMA,
            pltpu.SemaphoreType.DMA,
            pltpu.SemaphoreType.DMA,
            pltpu.SemaphoreType.DMA,
            pltpu.SemaphoreType.DMA,
        ],
        compiler_params=pltpu.CompilerParams(needs_layout_passes=False),
    )
    def emb_gather(tbl_hbm, idx_hbm, out_hbm,
                   idx0, idx1, idx2, idx3, gidx0, gidx1,
                   rows0, rows1, slab0, slab1,
                   isem, gsem0, gsem1, osem0, osem1):
        wid = lax.axis_index("s") * NUM_CORES + lax.axis_index("c")
        base = wid * BPW
        idxs = (idx0, idx1, idx2, idx3)
        gidxs = (gidx0, gidx1)
        rows = (rows0, rows1)
        slabs = (slab0, slab1)
        gsems = (gsem0, gsem1)
        osems = (osem0, osem1)
        lane = lax.broadcasted_iota(jnp.int32, (LANES,), 0)

        def idx_copy(c, slot):
            return pltpu.make_async_copy(
                idx_hbm.at[pl.ds(base + c * CHUNK, CHUNK)], idxs[slot], isem)

        def start_gather(slot, p):
            src = idxs[slot]
            for j in range(CHUNK // LANES):
                sl = pl.ds(j * LANES, LANES)
                gidxs[p][sl] = lax.shift_right_logical(src[sl], 1)
            pltpu.async_copy(tbl_hbm.at[gidxs[p]], rows[p], gsems[p])

        def wait_gather(p):
            pltpu.make_async_copy(
                tbl_hbm.at[gidxs[p]], rows[p], gsems[p]).wait()

        def slab_copy(c, p):
            return pltpu.make_async_copy(
                slabs[p],
                out_hbm.at[:, pl.ds(base + c * CHUNK, CHUNK)], osems[p])

        def transpose_chunk(slot, p):
            src = idxs[slot]
            for g in range(CHUNK // LANES):
                sl = pl.ds(g * LANES, LANES)
                half = lax.shift_left(
                    lax.bitwise_and(src[sl], jnp.int32(1)), 6)
                lvec = lane + g * LANES

                @plsc.parallel_loop(0, EMB_DIM, 1, unroll=8)
                def dloop(d):
                    cvec = lax.bitwise_and(lane + d, jnp.int32(EMB_DIM - 1))
                    vals = plsc.load_gather(rows[p], [lvec, half + cvec])
                    plsc.store_scatter(slabs[p], [cvec, lvec], vals)

        def phase(c, k, first):
            p = k % 2

            @pl.when(c + 1 < NCHUNKS)
            def _():
                idx_copy(c + 1, (k + 1) % 4).wait()
                start_gather((k + 1) % 4, 1 - p)

            @pl.when(c + 2 < NCHUNKS)
            def _():
                idx_copy(c + 2, (k + 2) % 4).start()

            wait_gather(p)
            if not first:
                @pl.when(c >= 2)
                def _():
                    slab_copy(c - 2, p).wait()

            transpose_chunk(k, p)
            slab_copy(c, p).start()

        idx_copy(0, 0).start()
        idx_copy(0, 0).wait()
        start_gather(0, 0)
        idx_copy(1, 1).start()

        def body(i, carry):
            c0 = 4 * i
            for k in range(4):
                phase(c0 + k, k, first=False)
            return carry

        for k in range(4):
            phase(jnp.int32(k), k, first=k < 2)
        lax.fori_loop(1, NQUAD, body, 0)
        slab_copy(NCHUNKS - 2, 0).wait()
        slab_copy(NCHUNKS - 1, 1).wait()

    return emb_gather


_relayout = _build_phase1()
_emb_gather = _build_phase2()


def kernel(input, batch_sizes, table):
    table_t = table.T
    tail2 = table[VOCAB - TAIL_ROWS:].reshape(TAIL_ROWS // 2, 128)
    tbl2 = _relayout(table_t, tail2)
    out_t = _emb_gather(tbl2, input)
    return (out_t.T, batch_sizes)
